# baseline (speedup 1.0000x reference)
import jax
import jax.numpy as jnp
from jax import lax
from jax.experimental import pallas as pl
from jax.experimental.pallas import tpu as pltpu

B, L, D_IN, D_H, D_OUT, TOPK = 4, 8192, 128, 64, 22, 8
NCHUNK = 8
CT = (B * L) // NCHUNK
CPB = L // CT


def _mlp_topk_kernel(x_hbm, w1_ref, b1_ref, w2_ref, b2_ref, o_hbm,
                     x_buf, y_buf, in_sems, out_sems):
    w1 = w1_ref[...]
    b1 = b1_ref[...]
    w2 = w2_ref[...]
    b2 = b2_ref[...]

    in_cps = []
    for i in range(NCHUNK):
        b, j = divmod(i, CPB)
        cp = pltpu.make_async_copy(
            x_hbm.at[b, pl.ds(j * CT, CT), :], x_buf.at[i], in_sems.at[i])
        cp.start()
        in_cps.append(cp)

    out_cps = []
    for i in range(NCHUNK):
        b, j = divmod(i, CPB)
        in_cps[i].wait()
        x = x_buf[i]
        h = lax.dot_general(x, w1, (((1,), (1,)), ((), ())),
                            preferred_element_type=jnp.float32)
        h = jnp.maximum(h + b1, 0.0)
        y = lax.dot_general(w2, h, (((1,), (1,)), ((), ())),
                            preferred_element_type=jnp.float32)
        y = jnp.maximum(y + b2, 0.0)
        cnt = jnp.zeros(y.shape, jnp.float32)
        for c in range(D_OUT):
            cnt = cnt + (y[c:c + 1, :] > y).astype(jnp.float32)
        y_buf[i] = jnp.where(cnt < float(TOPK), y, 0.0)
        cp = pltpu.make_async_copy(
            y_buf.at[i], o_hbm.at[b, :, pl.ds(j * CT, CT)], out_sems.at[i])
        cp.start()
        out_cps.append(cp)

    for cp in out_cps:
        cp.wait()


@jax.jit
def kernel(input, W1, b1, W2, b2):
    b1r = b1.reshape(1, D_H)
    b2r = b2.reshape(D_OUT, 1)
    return pl.pallas_call(
        _mlp_topk_kernel,
        in_specs=[
            pl.BlockSpec(memory_space=pl.ANY),
            pl.BlockSpec(memory_space=pltpu.MemorySpace.VMEM),
            pl.BlockSpec(memory_space=pltpu.MemorySpace.VMEM),
            pl.BlockSpec(memory_space=pltpu.MemorySpace.VMEM),
            pl.BlockSpec(memory_space=pltpu.MemorySpace.VMEM),
        ],
        out_specs=pl.BlockSpec(memory_space=pl.ANY),
        out_shape=jax.ShapeDtypeStruct((B, D_OUT, L), jnp.float32),
        scratch_shapes=[
            pltpu.VMEM((NCHUNK, CT, D_IN), jnp.float32),
            pltpu.VMEM((NCHUNK, D_OUT, CT), jnp.float32),
            pltpu.SemaphoreType.DMA((NCHUNK,)),
            pltpu.SemaphoreType.DMA((NCHUNK,)),
        ],
    )(input, W1, b1r, W2, b2r)


# speedup vs baseline: 1.1859x; 1.1859x over previous
import jax
import jax.numpy as jnp
from jax import lax
from jax.experimental import pallas as pl
from jax.experimental.pallas import tpu as pltpu

B, L, D_IN, D_H, D_OUT, TOPK = 4, 8192, 128, 64, 22, 8
NCHUNK = 8
WINDOW = 3
CT = (B * L) // NCHUNK
CPB = L // CT


def _mlp_topk_kernel(x_hbm, w1_ref, b1_ref, w2_ref, b2_ref, o_hbm,
                     x_buf, y_buf, in_sems, out_sems):
    w1 = w1_ref[...]
    b1 = b1_ref[...]
    w2 = w2_ref[...]
    b2 = b2_ref[...]

    def start_in(i):
        b, j = divmod(i, CPB)
        cp = pltpu.make_async_copy(
            x_hbm.at[b, pl.ds(j * CT, CT), :], x_buf.at[i], in_sems.at[i])
        cp.start()
        return cp

    in_cps = [start_in(i) for i in range(WINDOW)]

    out_cps = []
    for i in range(NCHUNK):
        b, j = divmod(i, CPB)
        in_cps[i].wait()
        if i + WINDOW < NCHUNK:
            in_cps.append(start_in(i + WINDOW))
        x = x_buf[i]
        h = lax.dot_general(x, w1, (((1,), (1,)), ((), ())),
                            preferred_element_type=jnp.float32)
        h = jnp.maximum(h + b1, 0.0)
        y = lax.dot_general(w2, h, (((1,), (1,)), ((), ())),
                            preferred_element_type=jnp.float32)
        y = jnp.maximum(y + b2, 0.0)
        yw = y
        for _ in range(TOPK - 1):
            m = jnp.max(yw, axis=0, keepdims=True)
            yw = jnp.where(yw >= m, -1.0, yw)
        t8 = jnp.max(yw, axis=0, keepdims=True)
        y_buf[i] = jnp.where(y >= t8, y, 0.0)
        cp = pltpu.make_async_copy(
            y_buf.at[i], o_hbm.at[b, :, pl.ds(j * CT, CT)], out_sems.at[i])
        cp.start()
        out_cps.append(cp)

    for cp in out_cps:
        cp.wait()


@jax.jit
def kernel(input, W1, b1, W2, b2):
    b1r = b1.reshape(1, D_H)
    b2r = b2.reshape(D_OUT, 1)
    return pl.pallas_call(
        _mlp_topk_kernel,
        in_specs=[
            pl.BlockSpec(memory_space=pl.ANY),
            pl.BlockSpec(memory_space=pltpu.MemorySpace.VMEM),
            pl.BlockSpec(memory_space=pltpu.MemorySpace.VMEM),
            pl.BlockSpec(memory_space=pltpu.MemorySpace.VMEM),
            pl.BlockSpec(memory_space=pltpu.MemorySpace.VMEM),
        ],
        out_specs=pl.BlockSpec(memory_space=pl.ANY),
        out_shape=jax.ShapeDtypeStruct((B, D_OUT, L), jnp.float32),
        scratch_shapes=[
            pltpu.VMEM((NCHUNK, CT, D_IN), jnp.float32),
            pltpu.VMEM((NCHUNK, D_OUT, CT), jnp.float32),
            pltpu.SemaphoreType.DMA((NCHUNK,)),
            pltpu.SemaphoreType.DMA((NCHUNK,)),
        ],
    )(input, W1, b1r, W2, b2r)
